# baseline (speedup 1.0000x reference)
import functools

import jax
import jax.numpy as jnp
from jax import lax
from jax.experimental import pallas as pl
from jax.experimental.pallas import tpu as pltpu
from jax.experimental.pallas import tpu_sc as plsc


def _sc_permute(z, perm, *, B, C, H, W, NC):
    BPC = B // NC

    mesh = plsc.ScalarSubcoreMesh(axis_name="core", num_cores=NC)

    @functools.partial(
        pl.kernel,
        mesh=mesh,
        out_type=jax.ShapeDtypeStruct((B, C, H, W), jnp.float32),
        compiler_params=pltpu.CompilerParams(use_tc_tiling_on_sc=True),
        scratch_types=[
            pltpu.SMEM((C,), jnp.int32),
            pltpu.SemaphoreType.DMA,
        ],
    )
    def scs_copy(z_hbm, perm_hbm, out_hbm, perm_s, sem):
        core = lax.axis_index("core")
        pltpu.sync_copy(perm_hbm, perm_s)
        base_b = core * BPC

        def issue_b(bi, _):
            b = base_b + bi

            def issue_c(c, _):
                pc = perm_s[c]
                pltpu.make_async_copy(
                    z_hbm.at[b, pc], out_hbm.at[b, c], sem
                ).start()
                return 0

            lax.fori_loop(0, C, issue_c, 0)
            return 0

        lax.fori_loop(0, BPC, issue_b, 0)

        def drain(i, _):
            pltpu.make_async_copy(
                z_hbm.at[0, 0], out_hbm.at[0, 0], sem
            ).wait()
            return 0

        lax.fori_loop(0, BPC * C, drain, 0)

    return scs_copy(z, perm)


def kernel(z, perm):
    B, C, H, W = z.shape
    info = plsc.get_sparse_core_info()
    out = _sc_permute(
        z, perm.astype(jnp.int32), B=B, C=C, H=H, W=W, NC=info.num_cores
    )
    return out, jnp.zeros((), z.dtype)


# speedup vs baseline: 39.8387x; 39.8387x over previous
import functools

import jax
import jax.numpy as jnp
from jax import lax
from jax.experimental import pallas as pl
from jax.experimental.pallas import tpu as pltpu
from jax.experimental.pallas import tpu_sc as plsc


def _sc_permute(z, perm, *, B, C, H, W, NC, NS):
    CHALF = C // 2

    mesh = plsc.VectorSubcoreMesh(core_axis_name="c", subcore_axis_name="s")

    @functools.partial(
        pl.kernel,
        mesh=mesh,
        out_type=jax.ShapeDtypeStruct((B, C, H, W), jnp.float32),
        compiler_params=pltpu.CompilerParams(
            use_tc_tiling_on_sc=True, needs_layout_passes=False
        ),
        scratch_types=[
            pltpu.VMEM((C,), jnp.int32),
            pltpu.VMEM((2, 1, 1, H, W), jnp.float32),
            pltpu.SemaphoreType.DMA,
            pltpu.SemaphoreType.DMA,
            pltpu.SemaphoreType.DMA,
            pltpu.SemaphoreType.DMA,
        ],
    )
    def sc_copy(z_hbm, perm_hbm, out_hbm, perm_v, buf_v, g0, g1, s0, s1):
        gsem = (g0, g1)
        ssem = (s0, s1)
        wid = lax.axis_index("s") * NC + lax.axis_index("c")
        pltpu.sync_copy(perm_hbm, perm_v)
        b = wid // 2
        c0 = (wid % 2) * CHALF

        def perm_at(c):
            vec = perm_v[pl.ds((c // 16) * 16, 16)]
            lane = lax.iota(jnp.int32, 16)
            return jnp.sum(jnp.where(lane == c % 16, vec, 0))

        def scatter_start(c, bb):
            pltpu.make_async_copy(
                buf_v.at[bb], out_hbm.at[pl.ds(b, 1), pl.ds(c, 1)], ssem[bb]
            ).start()

        def scatter_wait(bb):
            pltpu.make_async_copy(
                buf_v.at[bb], out_hbm.at[pl.ds(0, 1), pl.ds(0, 1)], ssem[bb]
            ).wait()

        def body(p, _):
            for bb in range(2):
                i = 2 * p + bb
                c = c0 + i

                @pl.when(p > 0)
                def _():
                    scatter_wait(bb)

                pc = perm_at(c)
                pltpu.async_copy(
                    z_hbm.at[pl.ds(b, 1), pl.ds(pc, 1)], buf_v.at[bb], gsem[bb]
                ).wait()
                scatter_start(c, bb)
            return 0

        lax.fori_loop(0, CHALF // 2, body, 0)
        scatter_wait(0)
        scatter_wait(1)

    return sc_copy(z, perm)


def kernel(z, perm):
    B, C, H, W = z.shape
    info = plsc.get_sparse_core_info()
    out = _sc_permute(
        z,
        perm.astype(jnp.int32),
        B=B,
        C=C,
        H=H,
        W=W,
        NC=info.num_cores,
        NS=info.num_subcores,
    )
    return out, jnp.zeros((), z.dtype)


# speedup vs baseline: 39.9750x; 1.0034x over previous
import functools

import jax
import jax.numpy as jnp
from jax import lax
from jax.experimental import pallas as pl
from jax.experimental.pallas import tpu as pltpu
from jax.experimental.pallas import tpu_sc as plsc

_NBUF = 8
_CPP = 4


def _sc_permute(z, perm, *, B, C, H, W, NC, NS):
    CHALF = C // 2
    CH = H // _CPP
    NCHUNK = CHALF * _CPP
    LOOK = _NBUF // 2

    mesh = plsc.VectorSubcoreMesh(core_axis_name="c", subcore_axis_name="s")

    @functools.partial(
        pl.kernel,
        mesh=mesh,
        out_type=jax.ShapeDtypeStruct((B, C, H, W), jnp.float32),
        compiler_params=pltpu.CompilerParams(
            use_tc_tiling_on_sc=True, needs_layout_passes=False
        ),
        scratch_types=[
            pltpu.VMEM((C,), jnp.int32),
            pltpu.VMEM((_NBUF, CH, W), jnp.float32),
            pltpu.SemaphoreType.DMA((_NBUF,)),
            pltpu.SemaphoreType.DMA((_NBUF,)),
        ],
    )
    def sc_copy(z_hbm, perm_hbm, out_hbm, perm_v, buf_v, gsem, ssem):
        wid = lax.axis_index("s") * NC + lax.axis_index("c")
        pltpu.sync_copy(perm_hbm, perm_v)
        b = wid // 2
        c0 = (wid % 2) * CHALF

        def perm_at(c):
            vec = perm_v[pl.ds((c // 16) * 16, 16)]
            lane = lax.iota(jnp.int32, 16)
            return jnp.sum(jnp.where(lane == c % 16, vec, 0))

        def gather_start(j, bb):
            ci = j % _CPP
            pc = perm_at(c0 + j // _CPP)
            pltpu.make_async_copy(
                z_hbm.at[b, pc, pl.ds(ci * CH, CH)], buf_v.at[bb], gsem.at[bb]
            ).start()

        def gather_wait(bb):
            pltpu.make_async_copy(
                z_hbm.at[0, 0, pl.ds(0, CH)], buf_v.at[bb], gsem.at[bb]
            ).wait()

        def scatter_start(j, bb):
            ci = j % _CPP
            pltpu.make_async_copy(
                buf_v.at[bb],
                out_hbm.at[b, c0 + j // _CPP, pl.ds(ci * CH, CH)],
                ssem.at[bb],
            ).start()

        def scatter_wait(bb):
            pltpu.make_async_copy(
                buf_v.at[bb], out_hbm.at[0, 0, pl.ds(0, CH)], ssem.at[bb]
            ).wait()

        for k in range(LOOK):
            gather_start(k, k)

        def body(p, _):
            for bb in range(_NBUF):
                j = _NBUF * p + bb
                gather_wait(bb)
                scatter_start(j, bb)
                jn = j + LOOK
                bn = (bb + LOOK) % _NBUF

                @pl.when(jn < NCHUNK)
                def _():
                    @pl.when(j >= LOOK)
                    def _():
                        scatter_wait(bn)

                    gather_start(jn, bn)

            return 0

        lax.fori_loop(0, NCHUNK // _NBUF, body, 0)
        for k in range(_NBUF):
            scatter_wait(k)

    return sc_copy(z, perm)


def kernel(z, perm):
    B, C, H, W = z.shape
    info = plsc.get_sparse_core_info()
    out = _sc_permute(
        z,
        perm.astype(jnp.int32),
        B=B,
        C=C,
        H=H,
        W=W,
        NC=info.num_cores,
        NS=info.num_subcores,
    )
    return out, jnp.zeros((), z.dtype)
